# Initial kernel scaffold; baseline (speedup 1.0000x reference)
import jax
import jax.numpy as jnp
from jax.experimental import pallas as pl


def kernel(mixture, w_enc, boundaries, prior0, prior1, lik, decode_table):
    raise NotImplementedError("write your pallas kernel here")


# speedup vs baseline: 58.6347x; 58.6347x over previous
import jax
import jax.numpy as jnp
from jax.experimental import pallas as pl
from jax.experimental.pallas import tpu as pltpu

_K = 256
_T = 512
_HOP = 128
_B = 8
_NEG = float("-inf")

_INTERPRET = False


def _encode_body(frames_ref, w_ref, bnd_ref, codes_ref):
    feats = jnp.dot(frames_ref[...], w_ref[...],
                    preferred_element_type=jnp.float32)
    cnt = jnp.sum((bnd_ref[...] < feats).astype(jnp.int32), axis=1, keepdims=True)
    codes_ref[...] = jnp.broadcast_to(cnt, codes_ref.shape)


def _scan_body(codes_ref, prior0_ref, prior1_ref, lik_ref,
               bps_ref, t0_ref, t1_ref,
               scores_smem, x0_smem, x1_smem, a_scr, p1_scr):
    t = pl.program_id(0)
    lane128 = jax.lax.broadcasted_iota(jnp.int32, (1, 128), 1)
    laneK = jax.lax.broadcasted_iota(jnp.int32, (1, _K), 1)

    def emit(winners, row):
        accb = jnp.zeros((1, 128), jnp.int32)
        acci = jnp.zeros((1, 128), jnp.int32)
        accj = jnp.zeros((1, 128), jnp.int32)
        for k, (_, b_, i_, j_) in enumerate(winners):
            accb = jnp.where(lane128 == k, b_, accb)
            acci = jnp.where(lane128 == k, i_, acci)
            accj = jnp.where(lane128 == k, j_, accj)
        bps_ref[pl.ds(row, 1), :] = accb
        t0_ref[pl.ds(row, 1), :] = acci
        t1_ref[pl.ds(row, 1), :] = accj
        for k, (v_, b_, i_, j_) in enumerate(winners):
            scores_smem[k] = v_
            x0_smem[k] = i_
            x1_smem[k] = j_

    @pl.when(t == 0)
    def _init():
        p0row = prior0_ref[0:1, :]
        p1row = prior1_ref[0:1, :]
        p0col = jnp.transpose(p0row)
        tmp = (p0col + p1row) + lik_ref[0]
        rowmax = jnp.max(tmp, axis=1, keepdims=True)
        icol = jax.lax.broadcasted_iota(jnp.int32, (_K, 1), 0)
        winners = []
        for k in range(_B):
            m = jnp.max(rowmax)
            i_ = jnp.min(jnp.where(rowmax == m, icol, _K))
            lrow = lik_ref[0, pl.ds(i_, 1), :]
            p0s = jnp.sum(jnp.where(laneK == i_, p0row, 0.0))
            rowv = (p0s + p1row) + lrow
            for (_, _, pi, pj) in winners:
                rowv = jnp.where((pi == i_) & (laneK == pj), _NEG, rowv)
            mv = jnp.max(rowv)
            j_ = jnp.min(jnp.where(rowv == mv, laneK, _K))
            nm = jnp.max(jnp.where(laneK == j_, _NEG, rowv))
            rowmax = jnp.where(icol == i_, nm, rowmax)
            winners.append((mv, k, i_, j_))
        emit(winners, 0)

    @pl.when(t > 0)
    def _step():
        s = [scores_smem[b] for b in range(_B)]
        p1rows = []
        for b in range(_B):
            p0r = prior0_ref[pl.ds(x0_smem[b], 1), :]
            p1r = prior1_ref[pl.ds(x1_smem[b], 1), :]
            a_scr[pl.ds(b, 1), :] = s[b] + p0r
            p1_scr[pl.ds(b, 1), :] = p1r
            p1rows.append(p1r)
        A = a_scr[...]
        AT = jnp.transpose(A)
        L = lik_ref[0]
        rmcols = []
        for b in range(_B):
            tmp = (AT[:, b:b + 1] + p1rows[b]) + L
            rmcols.append(jnp.max(tmp, axis=1, keepdims=True))
        rowmaxT = jnp.concatenate(rmcols, axis=1)
        icol = jax.lax.broadcasted_iota(jnp.int32, (_K, _B), 0)
        brow = jax.lax.broadcasted_iota(jnp.int32, (_K, _B), 1)
        flatT = brow * _K + icol
        winners = []
        for k in range(_B):
            m = jnp.max(rowmaxT)
            idx = jnp.min(jnp.where(rowmaxT == m, flatT, _B * _K))
            b_ = idx // _K
            i_ = idx % _K
            arow = a_scr[pl.ds(b_, 1), :]
            base = jnp.sum(jnp.where(laneK == i_, arow, 0.0))
            p1r = p1_scr[pl.ds(b_, 1), :]
            lrow = lik_ref[0, pl.ds(i_, 1), :]
            rowv = (base + p1r) + lrow
            for (_, pb, pi, pj) in winners:
                rowv = jnp.where((pb == b_) & (pi == i_) & (laneK == pj), _NEG, rowv)
            mv = jnp.max(rowv)
            j_ = jnp.min(jnp.where(rowv == mv, laneK, _K))
            nm = jnp.max(jnp.where(laneK == j_, _NEG, rowv))
            rowmaxT = jnp.where((icol == i_) & (brow == b_), nm, rowmaxT)
            winners.append((mv, b_, i_, j_))
        emit(winners, t)


def _backtrack_body(bps_ref, t0_ref, t1_ref, x0_ref, x1_ref):
    lane128 = jax.lax.broadcasted_iota(jnp.int32, (1, 128), 1)
    beam0 = jax.lax.broadcasted_iota(jnp.int32, (_B, 1), 0)

    def chunk(rbase, beam):
        def body(m, carry):
            beam, acc0, acc1 = carry
            r = rbase + 127 - m
            bprow = bps_ref[pl.ds(r, 1), :]
            t0row = t0_ref[pl.ds(r, 1), :]
            t1row = t1_ref[pl.ds(r, 1), :]
            oh = beam == lane128
            v0 = jnp.sum(jnp.where(oh, t0row, 0), axis=1, keepdims=True)
            v1 = jnp.sum(jnp.where(oh, t1row, 0), axis=1, keepdims=True)
            nb = jnp.sum(jnp.where(oh, bprow, 0), axis=1, keepdims=True)
            pos = 127 - m
            acc0 = jnp.where(lane128 == pos, v0, acc0)
            acc1 = jnp.where(lane128 == pos, v1, acc1)
            return (nb, acc0, acc1)

        z = jnp.zeros((_B, 128), jnp.int32)
        beam, acc0, acc1 = jax.lax.fori_loop(0, 128, body, (beam, z, z))
        x0_ref[:, rbase:rbase + 128] = acc0
        x1_ref[:, rbase:rbase + 128] = acc1
        return beam

    beam = beam0
    for rbase in range(_T - 128, -1, -128):
        beam = chunk(rbase, beam)


def _decode_body(tok_ref, table_ref, out_ref):
    cidx = jax.lax.broadcasted_iota(jnp.int32, (1, _K), 1)
    oh = (tok_ref[...] == cidx).astype(jnp.float32)
    out_ref[...] = jnp.dot(oh, table_ref[...],
                           precision=jax.lax.Precision.HIGHEST,
                           preferred_element_type=jnp.float32)


def _run_encode(mixture, w_enc, boundaries):
    frames = mixture.reshape(_T, _HOP)
    w2 = w_enc.reshape(_HOP, 1)
    bnd = jnp.concatenate(
        [boundaries, jnp.full((1,), jnp.inf, jnp.float32)]).reshape(1, _K)
    codes2d = pl.pallas_call(
        _encode_body,
        out_shape=jax.ShapeDtypeStruct((_T, 128), jnp.int32),
        interpret=_INTERPRET,
    )(frames, w2, bnd)
    return codes2d[:, 0]


def _run_scan(codes, prior0, prior1, lik):
    return pl.pallas_call(
        _scan_body,
        grid_spec=pltpu.PrefetchScalarGridSpec(
            num_scalar_prefetch=1,
            grid=(_T,),
            in_specs=[
                pl.BlockSpec((_K, _K), lambda t, c: (0, 0)),
                pl.BlockSpec((_K, _K), lambda t, c: (0, 0)),
                pl.BlockSpec((1, _K, _K), lambda t, c: (c[t], 0, 0)),
            ],
            out_specs=[
                pl.BlockSpec((_T, 128), lambda t, c: (0, 0)),
                pl.BlockSpec((_T, 128), lambda t, c: (0, 0)),
                pl.BlockSpec((_T, 128), lambda t, c: (0, 0)),
            ],
            scratch_shapes=[
                pltpu.SMEM((_B,), jnp.float32),
                pltpu.SMEM((_B,), jnp.int32),
                pltpu.SMEM((_B,), jnp.int32),
                pltpu.VMEM((_B, _K), jnp.float32),
                pltpu.VMEM((_B, _K), jnp.float32),
            ],
        ),
        out_shape=[jax.ShapeDtypeStruct((_T, 128), jnp.int32)] * 3,
        interpret=_INTERPRET,
    )(codes, prior0, prior1, lik)


def _run_backtrack(bps, t0s, t1s):
    return pl.pallas_call(
        _backtrack_body,
        out_shape=[jax.ShapeDtypeStruct((_B, _T), jnp.int32)] * 2,
        interpret=_INTERPRET,
    )(bps, t0s, t1s)


def _run_decode(x0s, x1s, decode_table):
    toks = jnp.concatenate([x0s.reshape(-1), x1s.reshape(-1)]).reshape(-1, 1)
    n = toks.shape[0]
    dec = pl.pallas_call(
        _decode_body,
        grid=(n // 512,),
        in_specs=[
            pl.BlockSpec((512, 1), lambda g: (g, 0)),
            pl.BlockSpec((_K, _HOP), lambda g: (0, 0)),
        ],
        out_specs=pl.BlockSpec((512, _HOP), lambda g: (g, 0)),
        out_shape=jax.ShapeDtypeStruct((n, _HOP), jnp.float32),
        interpret=_INTERPRET,
    )(toks, decode_table)

    half = n // 2
    d0 = dec[:half].reshape(_B, _T * _HOP)
    d1 = dec[half:].reshape(_B, _T * _HOP)
    return (d0, d1)


def kernel(mixture, w_enc, boundaries, prior0, prior1, lik, decode_table):
    codes = _run_encode(mixture, w_enc, boundaries)
    bps, t0s, t1s = _run_scan(codes, prior0, prior1, lik)
    x0s, x1s = _run_backtrack(bps, t0s, t1s)
    return _run_decode(x0s, x1s, decode_table)


# speedup vs baseline: 87.5360x; 1.4929x over previous
import functools

import jax
import jax.numpy as jnp
from jax.experimental import pallas as pl
from jax.experimental.pallas import tpu as pltpu
from jax.experimental.pallas import tpu_sc as plsc

_K = 256
_T = 512
_HOP = 128
_B = 8
_NEG = float("-inf")

_INTERPRET = False


def _encode_body(frames_ref, w_ref, bnd_ref, codes_ref):
    feats = jnp.dot(frames_ref[...], w_ref[...],
                    preferred_element_type=jnp.float32)
    cnt = jnp.sum((bnd_ref[...] < feats).astype(jnp.int32), axis=1, keepdims=True)
    codes_ref[...] = jnp.broadcast_to(cnt, codes_ref.shape)


def _scan_body(codes_ref, prior0_ref, prior1_ref, lik_ref,
               bps_ref, t0_ref, t1_ref,
               scores_smem, x0_smem, x1_smem, a_scr, p1_scr):
    t = pl.program_id(0)
    lane128 = jax.lax.broadcasted_iota(jnp.int32, (1, 128), 1)
    laneK = jax.lax.broadcasted_iota(jnp.int32, (1, _K), 1)

    def emit(winners, row):
        accb = jnp.zeros((1, 128), jnp.int32)
        acci = jnp.zeros((1, 128), jnp.int32)
        accj = jnp.zeros((1, 128), jnp.int32)
        for k, (_, b_, i_, j_) in enumerate(winners):
            accb = jnp.where(lane128 == k, b_, accb)
            acci = jnp.where(lane128 == k, i_, acci)
            accj = jnp.where(lane128 == k, j_, accj)
        bps_ref[pl.ds(row, 1), :] = accb
        t0_ref[pl.ds(row, 1), :] = acci
        t1_ref[pl.ds(row, 1), :] = accj
        for k, (v_, b_, i_, j_) in enumerate(winners):
            scores_smem[k] = v_
            x0_smem[k] = i_
            x1_smem[k] = j_

    @pl.when(t == 0)
    def _init():
        p0row = prior0_ref[0:1, :]
        p1row = prior1_ref[0:1, :]
        p0col = jnp.transpose(p0row)
        tmp = (p0col + p1row) + lik_ref[0]
        rowmax = jnp.transpose(jnp.max(tmp, axis=1, keepdims=True))
        winners = []
        for k in range(_B):
            m = jnp.max(rowmax)
            i_ = jnp.min(jnp.where(rowmax == m, laneK, _K))
            lrow = lik_ref[0, pl.ds(i_, 1), :]
            p0s = jnp.sum(jnp.where(laneK == i_, p0row, 0.0))
            rowv = (p0s + p1row) + lrow
            for (_, _, pi, pj) in winners:
                rowv = jnp.where((pi == i_) & (laneK == pj), _NEG, rowv)
            mv = jnp.max(rowv)
            j_ = jnp.min(jnp.where(rowv == mv, laneK, _K))
            nm = jnp.max(jnp.where(laneK == j_, _NEG, rowv))
            rowmax = jnp.where(laneK == i_, nm, rowmax)
            winners.append((mv, k, i_, j_))
        emit(winners, 0)

    @pl.when(t > 0)
    def _step():
        s = [scores_smem[b] for b in range(_B)]
        p1rows = []
        for b in range(_B):
            p0r = prior0_ref[pl.ds(x0_smem[b], 1), :]
            p1r = prior1_ref[pl.ds(x1_smem[b], 1), :]
            a_scr[pl.ds(b, 1), :] = s[b] + p0r
            p1_scr[pl.ds(b, 1), :] = p1r
            p1rows.append(p1r)
        A = a_scr[...]
        AT = jnp.transpose(A)
        L = lik_ref[0]
        rmcols = []
        for b in range(_B):
            tmp = (AT[:, b:b + 1] + p1rows[b]) + L
            rmcols.append(jnp.max(tmp, axis=1, keepdims=True))
        rowmax = jnp.transpose(jnp.concatenate(rmcols, axis=1))
        brow = jax.lax.broadcasted_iota(jnp.int32, (_B, _K), 0)
        icol = jax.lax.broadcasted_iota(jnp.int32, (_B, _K), 1)
        flat = brow * _K + icol

        def vmax2(x):
            return jnp.max(jnp.max(x, axis=1, keepdims=True),
                           axis=0, keepdims=True)

        def vmin2(x):
            return jnp.min(jnp.min(x, axis=1, keepdims=True),
                           axis=0, keepdims=True)

        rowsel = []
        for k in range(_B):
            m = vmax2(rowmax)
            fidx = vmin2(jnp.where(rowmax == m, flat, _B * _K))
            rowmax = jnp.where(flat == fidx, _NEG, rowmax)
            rowsel.append(fidx)

        rows = []
        rowflats = []
        for k in range(_B):
            ridx = jnp.sum(rowsel[k])
            b_ = ridx // _K
            i_ = ridx % _K
            arow = a_scr[pl.ds(b_, 1), :]
            base = jnp.sum(jnp.where(laneK == i_, arow, 0.0))
            p1r = p1_scr[pl.ds(b_, 1), :]
            lrow = lik_ref[0, pl.ds(i_, 1), :]
            rows.append((base + p1r) + lrow)
            rowflats.append(jnp.broadcast_to(ridx * _K, (1, _K)) + laneK)
        RR = jnp.concatenate(rows, axis=0)
        FF = jnp.concatenate(rowflats, axis=0)

        wins_v = []
        wins_f = []
        for k in range(_B):
            m = vmax2(RR)
            fidx = vmin2(jnp.where(RR == m, FF, _B * _K * _K))
            RR = jnp.where(FF == fidx, _NEG, RR)
            wins_v.append(m)
            wins_f.append(fidx)

        winners = []
        for k in range(_B):
            v_ = jnp.sum(wins_v[k])
            f_ = jnp.sum(wins_f[k])
            b_ = f_ // (_K * _K)
            i_ = (f_ // _K) % _K
            j_ = f_ % _K
            winners.append((v_, b_, i_, j_))
        emit(winners, t)


def _backtrack_body(bps_ref, t0_ref, t1_ref, x0_ref, x1_ref):
    lane128 = jax.lax.broadcasted_iota(jnp.int32, (1, 128), 1)
    beam0 = jax.lax.broadcasted_iota(jnp.int32, (_B, 1), 0)

    def chunk(rbase, beam):
        def body(m, carry):
            beam, acc0, acc1 = carry
            r = rbase + 127 - m
            bprow = bps_ref[pl.ds(r, 1), :]
            t0row = t0_ref[pl.ds(r, 1), :]
            t1row = t1_ref[pl.ds(r, 1), :]
            oh = beam == lane128
            v0 = jnp.sum(jnp.where(oh, t0row, 0), axis=1, keepdims=True)
            v1 = jnp.sum(jnp.where(oh, t1row, 0), axis=1, keepdims=True)
            nb = jnp.sum(jnp.where(oh, bprow, 0), axis=1, keepdims=True)
            pos = 127 - m
            acc0 = jnp.where(lane128 == pos, v0, acc0)
            acc1 = jnp.where(lane128 == pos, v1, acc1)
            return (nb, acc0, acc1)

        z = jnp.zeros((_B, 128), jnp.int32)
        beam, acc0, acc1 = jax.lax.fori_loop(0, 128, body, (beam, z, z))
        x0_ref[:, rbase:rbase + 128] = acc0
        x1_ref[:, rbase:rbase + 128] = acc1
        return beam

    beam = beam0
    for rbase in range(_T - 128, -1, -128):
        beam = chunk(rbase, beam)


def _decode_body(tok_ref, table_ref, out_ref):
    cidx = jax.lax.broadcasted_iota(jnp.int32, (1, _K), 1)
    oh = (tok_ref[...] == cidx).astype(jnp.float32)
    out_ref[...] = jnp.dot(oh, table_ref[...],
                           precision=jax.lax.Precision.HIGHEST,
                           preferred_element_type=jnp.float32)


def _run_encode(mixture, w_enc, boundaries):
    frames = mixture.reshape(_T, _HOP)
    w2 = w_enc.reshape(_HOP, 1)
    bnd = jnp.concatenate(
        [boundaries, jnp.full((1,), jnp.inf, jnp.float32)]).reshape(1, _K)
    codes2d = pl.pallas_call(
        _encode_body,
        out_shape=jax.ShapeDtypeStruct((_T, 128), jnp.int32),
        interpret=_INTERPRET,
    )(frames, w2, bnd)
    return codes2d[:, 0]


def _run_scan(codes, prior0, prior1, lik):
    return pl.pallas_call(
        _scan_body,
        grid_spec=pltpu.PrefetchScalarGridSpec(
            num_scalar_prefetch=1,
            grid=(_T,),
            in_specs=[
                pl.BlockSpec((_K, _K), lambda t, c: (0, 0)),
                pl.BlockSpec((_K, _K), lambda t, c: (0, 0)),
                pl.BlockSpec((1, _K, _K), lambda t, c: (c[t], 0, 0)),
            ],
            out_specs=[
                pl.BlockSpec((_T, 128), lambda t, c: (0, 0)),
                pl.BlockSpec((_T, 128), lambda t, c: (0, 0)),
                pl.BlockSpec((_T, 128), lambda t, c: (0, 0)),
            ],
            scratch_shapes=[
                pltpu.SMEM((_B,), jnp.float32),
                pltpu.SMEM((_B,), jnp.int32),
                pltpu.SMEM((_B,), jnp.int32),
                pltpu.VMEM((_B, _K), jnp.float32),
                pltpu.VMEM((_B, _K), jnp.float32),
            ],
        ),
        out_shape=[jax.ShapeDtypeStruct((_T, 128), jnp.int32)] * 3,
        interpret=_INTERPRET,
    )(codes, prior0, prior1, lik)


def _run_backtrack(bps, t0s, t1s):
    return pl.pallas_call(
        _backtrack_body,
        out_shape=[jax.ShapeDtypeStruct((_B, _T), jnp.int32)] * 2,
        interpret=_INTERPRET,
    )(bps, t0s, t1s)


def _run_decode(x0s, x1s, decode_table):
    toks = jnp.concatenate([x0s.reshape(-1), x1s.reshape(-1)]).reshape(-1, 1)
    n = toks.shape[0]
    dec = pl.pallas_call(
        _decode_body,
        grid=(n // 512,),
        in_specs=[
            pl.BlockSpec((512, 1), lambda g: (g, 0)),
            pl.BlockSpec((_K, _HOP), lambda g: (0, 0)),
        ],
        out_specs=pl.BlockSpec((512, _HOP), lambda g: (g, 0)),
        out_shape=jax.ShapeDtypeStruct((n, _HOP), jnp.float32),
        interpret=_INTERPRET,
    )(toks, decode_table)

    half = n // 2
    d0 = dec[:half].reshape(_B, _T * _HOP)
    d1 = dec[half:].reshape(_B, _T * _HOP)
    return (d0, d1)


def _sc_mesh():
    return plsc.VectorSubcoreMesh(core_axis_name="c", subcore_axis_name="s")


def _run_backtrack_sc(bps, t0s, t1s):
    bp = bps[:, :_B].reshape(-1)
    t0 = t0s[:, :_B].reshape(-1)
    t1 = t1s[:, :_B].reshape(-1)
    n = _B * _T
    npad = n + 16

    @functools.partial(
        pl.kernel, mesh=_sc_mesh(),
        out_type=[jax.ShapeDtypeStruct((16 * _T,), jnp.int32)] * 2,
        scratch_types=[pltpu.VMEM((npad,), jnp.int32)] * 3
        + [pltpu.VMEM((16 * _T,), jnp.int32)] * 2,
    )
    def bt(bp_hbm, t0_hbm, t1_hbm, x0_hbm, x1_hbm, bpv, t0v, t1v, x0v, x1v):
        wid = jax.lax.axis_index("s") * 2 + jax.lax.axis_index("c")

        @pl.when(wid == 0)
        def _():
            pltpu.sync_copy(bp_hbm, bpv.at[pl.ds(0, n)])
            pltpu.sync_copy(t0_hbm, t0v.at[pl.ds(0, n)])
            pltpu.sync_copy(t1_hbm, t1v.at[pl.ds(0, n)])
            lane = jax.lax.iota(jnp.int32, 16)
            msk = lane < _B
            beam0 = jnp.where(msk, lane, 0)

            def body(m, beam):
                r = (_T - 1) - m
                row0 = t0v[pl.ds(r * _B, 16)]
                row1 = t1v[pl.ds(r * _B, 16)]
                rowb = bpv[pl.ds(r * _B, 16)]
                x0v[pl.ds(r * 16, 16)] = row0.at[beam].get(
                    mode="promise_in_bounds")
                x1v[pl.ds(r * 16, 16)] = row1.at[beam].get(
                    mode="promise_in_bounds")
                nb = rowb.at[beam].get(mode="promise_in_bounds")
                return jnp.where(msk, nb, 0)

            jax.lax.fori_loop(0, _T, body, beam0)
            pltpu.sync_copy(x0v, x0_hbm)
            pltpu.sync_copy(x1v, x1_hbm)

    x0f, x1f = bt(bp, t0, t1)
    x0s = x0f.reshape(_T, 16)[:, :_B].T
    x1s = x1f.reshape(_T, 16)[:, :_B].T
    return x0s, x1s


def _run_decode_sc(x0s, x1s, decode_table):
    toks = jnp.concatenate([x0s.reshape(-1), x1s.reshape(-1)])
    n = toks.shape[0]
    nw = 32
    bpw = n // nw

    @functools.partial(
        pl.kernel, mesh=_sc_mesh(),
        out_type=jax.ShapeDtypeStruct((n, _HOP), jnp.float32),
        scratch_types=[
            pltpu.VMEM((bpw,), jnp.int32),
            pltpu.VMEM((bpw, _HOP), jnp.float32),
            pltpu.SemaphoreType.DMA,
        ],
    )
    def dec(tok_hbm, table_hbm, out_hbm, idx_v, rows_v, sem):
        wid = jax.lax.axis_index("s") * 2 + jax.lax.axis_index("c")
        base = wid * bpw
        pltpu.sync_copy(tok_hbm.at[pl.ds(base, bpw)], idx_v)
        pltpu.async_copy(table_hbm.at[idx_v], rows_v, sem).wait()
        pltpu.sync_copy(rows_v, out_hbm.at[pl.ds(base, bpw)])

    dec_out = dec(toks, decode_table)
    half = n // 2
    d0 = dec_out[:half].reshape(_B, _T * _HOP)
    d1 = dec_out[half:].reshape(_B, _T * _HOP)
    return (d0, d1)


def kernel(mixture, w_enc, boundaries, prior0, prior1, lik, decode_table):
    codes = _run_encode(mixture, w_enc, boundaries)
    bps, t0s, t1s = _run_scan(codes, prior0, prior1, lik)
    x0s, x1s = _run_backtrack_sc(bps, t0s, t1s)
    return _run_decode_sc(x0s, x1s, decode_table)


# speedup vs baseline: 115.1829x; 1.3158x over previous
import functools

import jax
import jax.numpy as jnp
from jax.experimental import pallas as pl
from jax.experimental.pallas import tpu as pltpu
from jax.experimental.pallas import tpu_sc as plsc

_K = 256
_T = 512
_HOP = 128
_B = 8
_NEG = float("-inf")

_INTERPRET = False


def _encode_body(frames_ref, w_ref, bnd_ref, codes_ref):
    feats = jnp.dot(frames_ref[...], w_ref[...],
                    preferred_element_type=jnp.float32)
    cnt = jnp.sum((bnd_ref[...] < feats).astype(jnp.int32), axis=1, keepdims=True)
    codes_ref[...] = jnp.broadcast_to(cnt, codes_ref.shape)


def _scan_body(codes_ref, prior0_ref, prior1_ref, lik_ref,
               bps_ref, t0_ref, t1_ref,
               scores_smem, x0_smem, x1_smem, a_scr, p1_scr):
    t = pl.program_id(0)
    lane128 = jax.lax.broadcasted_iota(jnp.int32, (1, 128), 1)
    laneK = jax.lax.broadcasted_iota(jnp.int32, (1, _K), 1)

    def emit(winners, row):
        accb = jnp.zeros((1, 128), jnp.int32)
        acci = jnp.zeros((1, 128), jnp.int32)
        accj = jnp.zeros((1, 128), jnp.int32)
        for k, (_, b_, i_, j_) in enumerate(winners):
            accb = jnp.where(lane128 == k, b_, accb)
            acci = jnp.where(lane128 == k, i_, acci)
            accj = jnp.where(lane128 == k, j_, accj)
        bps_ref[pl.ds(row, 1), :] = accb
        t0_ref[pl.ds(row, 1), :] = acci
        t1_ref[pl.ds(row, 1), :] = accj
        for k, (v_, b_, i_, j_) in enumerate(winners):
            scores_smem[k] = v_
            x0_smem[k] = i_
            x1_smem[k] = j_

    @pl.when(t == 0)
    def _init():
        p0row = prior0_ref[0:1, :]
        p1row = prior1_ref[0:1, :]
        p0col = jnp.transpose(p0row)
        tmp = (p0col + p1row) + lik_ref[0]
        rowmax = jnp.transpose(jnp.max(tmp, axis=1, keepdims=True))
        winners = []
        for k in range(_B):
            m = jnp.max(rowmax)
            i_ = jnp.min(jnp.where(rowmax == m, laneK, _K))
            lrow = lik_ref[0, pl.ds(i_, 1), :]
            p0s = jnp.sum(jnp.where(laneK == i_, p0row, 0.0))
            rowv = (p0s + p1row) + lrow
            for (_, _, pi, pj) in winners:
                rowv = jnp.where((pi == i_) & (laneK == pj), _NEG, rowv)
            mv = jnp.max(rowv)
            j_ = jnp.min(jnp.where(rowv == mv, laneK, _K))
            nm = jnp.max(jnp.where(laneK == j_, _NEG, rowv))
            rowmax = jnp.where(laneK == i_, nm, rowmax)
            winners.append((mv, k, i_, j_))
        emit(winners, 0)

    @pl.when(t > 0)
    def _step():
        s = [scores_smem[b] for b in range(_B)]
        p1rows = []
        for b in range(_B):
            p0r = prior0_ref[pl.ds(x0_smem[b], 1), :]
            p1r = prior1_ref[pl.ds(x1_smem[b], 1), :]
            a_scr[pl.ds(b, 1), :] = s[b] + p0r
            p1_scr[pl.ds(b, 1), :] = p1r
            p1rows.append(p1r)
        A = a_scr[...]
        AT = jnp.transpose(A)
        L = lik_ref[0]
        rmcols = []
        for b in range(_B):
            tmp = (AT[:, b:b + 1] + p1rows[b]) + L
            rmcols.append(jnp.max(tmp, axis=1, keepdims=True))
        rowmax0 = jnp.transpose(jnp.concatenate(rmcols, axis=1))
        brow = jax.lax.broadcasted_iota(jnp.int32, (_B, _K), 0)
        icol = jax.lax.broadcasted_iota(jnp.int32, (_B, _K), 1)
        flat = brow * _K + icol

        def vmax2(x):
            return jnp.max(jnp.max(x, axis=1, keepdims=True),
                           axis=0, keepdims=True)

        def vmin2(x):
            return jnp.min(jnp.min(x, axis=1, keepdims=True),
                           axis=0, keepdims=True)

        def gather_row(ridx):
            b_ = ridx // _K
            i_ = ridx % _K
            arow = a_scr[pl.ds(b_, 1), :]
            base = jnp.sum(jnp.where(laneK == i_, arow, 0.0))
            p1r = p1_scr[pl.ds(b_, 1), :]
            lrow = lik_ref[0, pl.ds(i_, 1), :]
            return (base + p1r) + lrow

        rm = rowmax0
        ms = []
        for k in range(_B):
            m = vmax2(rm)
            ms.append(m)
            rm = jnp.where(rm == m, _NEG, rm)
        fidxs = [vmin2(jnp.where(rowmax0 == ms[k], flat, _B * _K))
                 for k in range(_B)]
        c1 = sum(jnp.sum(jnp.where(rowmax0 == ms[k], 1.0, 0.0))
                 for k in range(_B))

        rows = []
        rowflats = []
        for k in range(_B):
            ridx = jnp.sum(fidxs[k])
            rows.append(gather_row(ridx))
            rowflats.append(jnp.broadcast_to(ridx * _K, (1, _K)) + laneK)
        RR = jnp.concatenate(rows, axis=0)
        FF = jnp.concatenate(rowflats, axis=0)

        rr = RR
        m3 = []
        for k in range(_B):
            m = vmax2(rr)
            m3.append(m)
            rr = jnp.where(rr == m, _NEG, rr)
        f3 = [vmin2(jnp.where(RR == m3[k], FF, _B * _K * _K))
              for k in range(_B)]
        c3 = sum(jnp.sum(jnp.where(RR == m3[k], 1.0, 0.0))
                 for k in range(_B))

        winners = []
        for k in range(_B):
            v_ = jnp.sum(m3[k])
            f_ = jnp.sum(f3[k])
            b_ = f_ // (_K * _K)
            i_ = (f_ // _K) % _K
            j_ = f_ % _K
            winners.append((v_, b_, i_, j_))
        emit(winners, t)

        @pl.when((c1 != 8.0) | (c3 != 8.0))
        def _slow():
            rowmax = rowmax0
            rowsel = []
            for k in range(_B):
                m = vmax2(rowmax)
                fidx = vmin2(jnp.where(rowmax == m, flat, _B * _K))
                rowmax = jnp.where(flat == fidx, _NEG, rowmax)
                rowsel.append(fidx)
            rows2 = []
            rowflats2 = []
            for k in range(_B):
                ridx = jnp.sum(rowsel[k])
                rows2.append(gather_row(ridx))
                rowflats2.append(
                    jnp.broadcast_to(ridx * _K, (1, _K)) + laneK)
            RR2 = jnp.concatenate(rows2, axis=0)
            FF2 = jnp.concatenate(rowflats2, axis=0)
            wins_v = []
            wins_f = []
            for k in range(_B):
                m = vmax2(RR2)
                fidx = vmin2(jnp.where(RR2 == m, FF2, _B * _K * _K))
                RR2 = jnp.where(FF2 == fidx, _NEG, RR2)
                wins_v.append(m)
                wins_f.append(fidx)
            winners2 = []
            for k in range(_B):
                v_ = jnp.sum(wins_v[k])
                f_ = jnp.sum(wins_f[k])
                b_ = f_ // (_K * _K)
                i_ = (f_ // _K) % _K
                j_ = f_ % _K
                winners2.append((v_, b_, i_, j_))
            emit(winners2, t)


def _backtrack_body(bps_ref, t0_ref, t1_ref, x0_ref, x1_ref):
    lane128 = jax.lax.broadcasted_iota(jnp.int32, (1, 128), 1)
    beam0 = jax.lax.broadcasted_iota(jnp.int32, (_B, 1), 0)

    def chunk(rbase, beam):
        def body(m, carry):
            beam, acc0, acc1 = carry
            r = rbase + 127 - m
            bprow = bps_ref[pl.ds(r, 1), :]
            t0row = t0_ref[pl.ds(r, 1), :]
            t1row = t1_ref[pl.ds(r, 1), :]
            oh = beam == lane128
            v0 = jnp.sum(jnp.where(oh, t0row, 0), axis=1, keepdims=True)
            v1 = jnp.sum(jnp.where(oh, t1row, 0), axis=1, keepdims=True)
            nb = jnp.sum(jnp.where(oh, bprow, 0), axis=1, keepdims=True)
            pos = 127 - m
            acc0 = jnp.where(lane128 == pos, v0, acc0)
            acc1 = jnp.where(lane128 == pos, v1, acc1)
            return (nb, acc0, acc1)

        z = jnp.zeros((_B, 128), jnp.int32)
        beam, acc0, acc1 = jax.lax.fori_loop(0, 128, body, (beam, z, z))
        x0_ref[:, rbase:rbase + 128] = acc0
        x1_ref[:, rbase:rbase + 128] = acc1
        return beam

    beam = beam0
    for rbase in range(_T - 128, -1, -128):
        beam = chunk(rbase, beam)


def _decode_body(tok_ref, table_ref, out_ref):
    cidx = jax.lax.broadcasted_iota(jnp.int32, (1, _K), 1)
    oh = (tok_ref[...] == cidx).astype(jnp.float32)
    out_ref[...] = jnp.dot(oh, table_ref[...],
                           precision=jax.lax.Precision.HIGHEST,
                           preferred_element_type=jnp.float32)


def _run_encode(mixture, w_enc, boundaries):
    frames = mixture.reshape(_T, _HOP)
    w2 = w_enc.reshape(_HOP, 1)
    bnd = jnp.concatenate(
        [boundaries, jnp.full((1,), jnp.inf, jnp.float32)]).reshape(1, _K)
    codes2d = pl.pallas_call(
        _encode_body,
        out_shape=jax.ShapeDtypeStruct((_T, 128), jnp.int32),
        interpret=_INTERPRET,
    )(frames, w2, bnd)
    return codes2d[:, 0]


def _run_scan(codes, prior0, prior1, lik):
    return pl.pallas_call(
        _scan_body,
        grid_spec=pltpu.PrefetchScalarGridSpec(
            num_scalar_prefetch=1,
            grid=(_T,),
            in_specs=[
                pl.BlockSpec((_K, _K), lambda t, c: (0, 0)),
                pl.BlockSpec((_K, _K), lambda t, c: (0, 0)),
                pl.BlockSpec((1, _K, _K), lambda t, c: (c[t], 0, 0)),
            ],
            out_specs=[
                pl.BlockSpec((_T, 128), lambda t, c: (0, 0)),
                pl.BlockSpec((_T, 128), lambda t, c: (0, 0)),
                pl.BlockSpec((_T, 128), lambda t, c: (0, 0)),
            ],
            scratch_shapes=[
                pltpu.SMEM((_B,), jnp.float32),
                pltpu.SMEM((_B,), jnp.int32),
                pltpu.SMEM((_B,), jnp.int32),
                pltpu.VMEM((_B, _K), jnp.float32),
                pltpu.VMEM((_B, _K), jnp.float32),
            ],
        ),
        out_shape=[jax.ShapeDtypeStruct((_T, 128), jnp.int32)] * 3,
        interpret=_INTERPRET,
    )(codes, prior0, prior1, lik)


def _run_backtrack(bps, t0s, t1s):
    return pl.pallas_call(
        _backtrack_body,
        out_shape=[jax.ShapeDtypeStruct((_B, _T), jnp.int32)] * 2,
        interpret=_INTERPRET,
    )(bps, t0s, t1s)


def _run_decode(x0s, x1s, decode_table):
    toks = jnp.concatenate([x0s.reshape(-1), x1s.reshape(-1)]).reshape(-1, 1)
    n = toks.shape[0]
    dec = pl.pallas_call(
        _decode_body,
        grid=(n // 512,),
        in_specs=[
            pl.BlockSpec((512, 1), lambda g: (g, 0)),
            pl.BlockSpec((_K, _HOP), lambda g: (0, 0)),
        ],
        out_specs=pl.BlockSpec((512, _HOP), lambda g: (g, 0)),
        out_shape=jax.ShapeDtypeStruct((n, _HOP), jnp.float32),
        interpret=_INTERPRET,
    )(toks, decode_table)

    half = n // 2
    d0 = dec[:half].reshape(_B, _T * _HOP)
    d1 = dec[half:].reshape(_B, _T * _HOP)
    return (d0, d1)


def _sc_mesh():
    return plsc.VectorSubcoreMesh(core_axis_name="c", subcore_axis_name="s")


def _run_backtrack_sc(bps, t0s, t1s):
    bp = bps[:, :_B].reshape(-1)
    t0 = t0s[:, :_B].reshape(-1)
    t1 = t1s[:, :_B].reshape(-1)
    n = _B * _T
    npad = n + 16

    @functools.partial(
        pl.kernel, mesh=_sc_mesh(),
        out_type=[jax.ShapeDtypeStruct((16 * _T,), jnp.int32)] * 2,
        scratch_types=[pltpu.VMEM((npad,), jnp.int32)] * 3
        + [pltpu.VMEM((16 * _T,), jnp.int32)] * 2,
    )
    def bt(bp_hbm, t0_hbm, t1_hbm, x0_hbm, x1_hbm, bpv, t0v, t1v, x0v, x1v):
        wid = jax.lax.axis_index("s") * 2 + jax.lax.axis_index("c")

        @pl.when(wid == 0)
        def _():
            pltpu.sync_copy(bp_hbm, bpv.at[pl.ds(0, n)])
            pltpu.sync_copy(t0_hbm, t0v.at[pl.ds(0, n)])
            pltpu.sync_copy(t1_hbm, t1v.at[pl.ds(0, n)])
            lane = jax.lax.iota(jnp.int32, 16)
            msk = lane < _B
            beam0 = jnp.where(msk, lane, 0)

            def body(m, beam):
                r = (_T - 1) - m
                row0 = t0v[pl.ds(r * _B, 16)]
                row1 = t1v[pl.ds(r * _B, 16)]
                rowb = bpv[pl.ds(r * _B, 16)]
                x0v[pl.ds(r * 16, 16)] = row0.at[beam].get(
                    mode="promise_in_bounds")
                x1v[pl.ds(r * 16, 16)] = row1.at[beam].get(
                    mode="promise_in_bounds")
                nb = rowb.at[beam].get(mode="promise_in_bounds")
                return jnp.where(msk, nb, 0)

            jax.lax.fori_loop(0, _T, body, beam0)
            pltpu.sync_copy(x0v, x0_hbm)
            pltpu.sync_copy(x1v, x1_hbm)

    x0f, x1f = bt(bp, t0, t1)
    x0s = x0f.reshape(_T, 16)[:, :_B].T
    x1s = x1f.reshape(_T, 16)[:, :_B].T
    return x0s, x1s


def _run_decode_sc(x0s, x1s, decode_table):
    toks = jnp.concatenate([x0s.reshape(-1), x1s.reshape(-1)])
    n = toks.shape[0]
    nw = 32
    bpw = n // nw

    @functools.partial(
        pl.kernel, mesh=_sc_mesh(),
        out_type=jax.ShapeDtypeStruct((n, _HOP), jnp.float32),
        scratch_types=[
            pltpu.VMEM((bpw,), jnp.int32),
            pltpu.VMEM((bpw, _HOP), jnp.float32),
            pltpu.SemaphoreType.DMA,
        ],
    )
    def dec(tok_hbm, table_hbm, out_hbm, idx_v, rows_v, sem):
        wid = jax.lax.axis_index("s") * 2 + jax.lax.axis_index("c")
        base = wid * bpw
        pltpu.sync_copy(tok_hbm.at[pl.ds(base, bpw)], idx_v)
        pltpu.async_copy(table_hbm.at[idx_v], rows_v, sem).wait()
        pltpu.sync_copy(rows_v, out_hbm.at[pl.ds(base, bpw)])

    dec_out = dec(toks, decode_table)
    half = n // 2
    d0 = dec_out[:half].reshape(_B, _T * _HOP)
    d1 = dec_out[half:].reshape(_B, _T * _HOP)
    return (d0, d1)


def kernel(mixture, w_enc, boundaries, prior0, prior1, lik, decode_table):
    codes = _run_encode(mixture, w_enc, boundaries)
    bps, t0s, t1s = _run_scan(codes, prior0, prior1, lik)
    x0s, x1s = _run_backtrack_sc(bps, t0s, t1s)
    return _run_decode_sc(x0s, x1s, decode_table)


# speedup vs baseline: 115.5855x; 1.0035x over previous
import functools

import jax
import jax.numpy as jnp
from jax.experimental import pallas as pl
from jax.experimental.pallas import tpu as pltpu
from jax.experimental.pallas import tpu_sc as plsc

_K = 256
_T = 512
_HOP = 128
_B = 8
_NEG = float("-inf")

_INTERPRET = False


def _encode_body(frames_ref, w_ref, bnd_ref, codes_ref):
    feats = jnp.dot(frames_ref[...], w_ref[...],
                    preferred_element_type=jnp.float32)
    cnt = jnp.sum((bnd_ref[...] < feats).astype(jnp.int32), axis=1, keepdims=True)
    codes_ref[...] = jnp.broadcast_to(cnt, codes_ref.shape)


def _scan_body(codes_ref, prior0_ref, prior1_ref, lik_ref,
               bps_ref, t0_ref, t1_ref,
               scores_smem, x0_smem, x1_smem, a_scr, p1_scr):
    t = pl.program_id(0)
    lane128 = jax.lax.broadcasted_iota(jnp.int32, (1, 128), 1)
    laneK = jax.lax.broadcasted_iota(jnp.int32, (1, _K), 1)

    def emit(winners, row):
        accb = jnp.zeros((1, 128), jnp.int32)
        acci = jnp.zeros((1, 128), jnp.int32)
        accj = jnp.zeros((1, 128), jnp.int32)
        for k, (_, b_, i_, j_) in enumerate(winners):
            accb = jnp.where(lane128 == k, b_, accb)
            acci = jnp.where(lane128 == k, i_, acci)
            accj = jnp.where(lane128 == k, j_, accj)
        bps_ref[pl.ds(row, 1), :] = accb
        t0_ref[pl.ds(row, 1), :] = acci
        t1_ref[pl.ds(row, 1), :] = accj
        for k, (v_, b_, i_, j_) in enumerate(winners):
            scores_smem[k] = v_
            x0_smem[k] = i_
            x1_smem[k] = j_

    @pl.when(t == 0)
    def _init():
        p0row = prior0_ref[0:1, :]
        p1row = prior1_ref[0:1, :]
        p0col = jnp.transpose(p0row)
        tmp = (p0col + p1row) + lik_ref[0]
        rowmax = jnp.transpose(jnp.max(tmp, axis=1, keepdims=True))
        winners = []
        for k in range(_B):
            m = jnp.max(rowmax)
            i_ = jnp.min(jnp.where(rowmax == m, laneK, _K))
            lrow = lik_ref[0, pl.ds(i_, 1), :]
            p0s = jnp.sum(jnp.where(laneK == i_, p0row, 0.0))
            rowv = (p0s + p1row) + lrow
            for (_, _, pi, pj) in winners:
                rowv = jnp.where((pi == i_) & (laneK == pj), _NEG, rowv)
            mv = jnp.max(rowv)
            j_ = jnp.min(jnp.where(rowv == mv, laneK, _K))
            nm = jnp.max(jnp.where(laneK == j_, _NEG, rowv))
            rowmax = jnp.where(laneK == i_, nm, rowmax)
            winners.append((mv, k, i_, j_))
        emit(winners, 0)

    @pl.when(t > 0)
    def _step():
        s = [scores_smem[b] for b in range(_B)]
        p1rows = []
        for b in range(_B):
            p0r = prior0_ref[pl.ds(x0_smem[b], 1), :]
            p1r = prior1_ref[pl.ds(x1_smem[b], 1), :]
            a_scr[pl.ds(b, 1), :] = s[b] + p0r
            p1_scr[pl.ds(b, 1), :] = p1r
            p1rows.append(p1r)
        A = a_scr[...]
        AT = jnp.transpose(A)
        L = lik_ref[0]
        rmcols = []
        for b in range(_B):
            tmp = (AT[:, b:b + 1] + p1rows[b]) + L
            rmcols.append(jnp.max(tmp, axis=1, keepdims=True))
        rowmax0 = jnp.transpose(jnp.concatenate(rmcols, axis=1))
        brow = jax.lax.broadcasted_iota(jnp.int32, (_B, _K), 0)
        icol = jax.lax.broadcasted_iota(jnp.int32, (_B, _K), 1)
        flat = brow * _K + icol

        def vmax2(x):
            return jnp.max(jnp.max(x, axis=1, keepdims=True),
                           axis=0, keepdims=True)

        def vmin2(x):
            return jnp.min(jnp.min(x, axis=1, keepdims=True),
                           axis=0, keepdims=True)

        def gather_row(ridx):
            b_ = ridx // _K
            i_ = ridx % _K
            arow = a_scr[pl.ds(b_, 1), :]
            base = jnp.sum(jnp.where(laneK == i_, arow, 0.0))
            p1r = p1_scr[pl.ds(b_, 1), :]
            lrow = lik_ref[0, pl.ds(i_, 1), :]
            return (base + p1r) + lrow

        rm = rowmax0
        ms = []
        for k in range(_B):
            m = vmax2(rm)
            ms.append(m)
            rm = jnp.where(rm == m, _NEG, rm)
        fidxs = [vmin2(jnp.where(rowmax0 == ms[k], flat, _B * _K))
                 for k in range(_B)]
        c1 = sum(jnp.sum(jnp.where(rowmax0 == ms[k], 1.0, 0.0))
                 for k in range(_B))

        rows = []
        rowflats = []
        for k in range(_B):
            ridx = jnp.sum(fidxs[k])
            rows.append(gather_row(ridx))
            rowflats.append(jnp.broadcast_to(ridx * _K, (1, _K)) + laneK)
        RR = jnp.concatenate(rows, axis=0)
        FF = jnp.concatenate(rowflats, axis=0)

        rr = RR
        m3 = []
        for k in range(_B):
            m = vmax2(rr)
            m3.append(m)
            rr = jnp.where(rr == m, _NEG, rr)
        f3 = [vmin2(jnp.where(RR == m3[k], FF, _B * _K * _K))
              for k in range(_B)]
        c3 = sum(jnp.sum(jnp.where(RR == m3[k], 1.0, 0.0))
                 for k in range(_B))

        winners = []
        for k in range(_B):
            v_ = jnp.sum(m3[k])
            f_ = jnp.sum(f3[k])
            b_ = f_ // (_K * _K)
            i_ = (f_ // _K) % _K
            j_ = f_ % _K
            winners.append((v_, b_, i_, j_))
        emit(winners, t)

        @pl.when((c1 != 8.0) | (c3 != 8.0))
        def _slow():
            rowmax = rowmax0
            rowsel = []
            for k in range(_B):
                m = vmax2(rowmax)
                fidx = vmin2(jnp.where(rowmax == m, flat, _B * _K))
                rowmax = jnp.where(flat == fidx, _NEG, rowmax)
                rowsel.append(fidx)
            rows2 = []
            rowflats2 = []
            for k in range(_B):
                ridx = jnp.sum(rowsel[k])
                rows2.append(gather_row(ridx))
                rowflats2.append(
                    jnp.broadcast_to(ridx * _K, (1, _K)) + laneK)
            RR2 = jnp.concatenate(rows2, axis=0)
            FF2 = jnp.concatenate(rowflats2, axis=0)
            wins_v = []
            wins_f = []
            for k in range(_B):
                m = vmax2(RR2)
                fidx = vmin2(jnp.where(RR2 == m, FF2, _B * _K * _K))
                RR2 = jnp.where(FF2 == fidx, _NEG, RR2)
                wins_v.append(m)
                wins_f.append(fidx)
            winners2 = []
            for k in range(_B):
                v_ = jnp.sum(wins_v[k])
                f_ = jnp.sum(wins_f[k])
                b_ = f_ // (_K * _K)
                i_ = (f_ // _K) % _K
                j_ = f_ % _K
                winners2.append((v_, b_, i_, j_))
            emit(winners2, t)


def _backtrack_body(bps_ref, t0_ref, t1_ref, x0_ref, x1_ref):
    lane128 = jax.lax.broadcasted_iota(jnp.int32, (1, 128), 1)
    beam0 = jax.lax.broadcasted_iota(jnp.int32, (_B, 1), 0)

    def chunk(rbase, beam):
        def body(m, carry):
            beam, acc0, acc1 = carry
            r = rbase + 127 - m
            bprow = bps_ref[pl.ds(r, 1), :]
            t0row = t0_ref[pl.ds(r, 1), :]
            t1row = t1_ref[pl.ds(r, 1), :]
            oh = beam == lane128
            v0 = jnp.sum(jnp.where(oh, t0row, 0), axis=1, keepdims=True)
            v1 = jnp.sum(jnp.where(oh, t1row, 0), axis=1, keepdims=True)
            nb = jnp.sum(jnp.where(oh, bprow, 0), axis=1, keepdims=True)
            pos = 127 - m
            acc0 = jnp.where(lane128 == pos, v0, acc0)
            acc1 = jnp.where(lane128 == pos, v1, acc1)
            return (nb, acc0, acc1)

        z = jnp.zeros((_B, 128), jnp.int32)
        beam, acc0, acc1 = jax.lax.fori_loop(0, 128, body, (beam, z, z))
        x0_ref[:, rbase:rbase + 128] = acc0
        x1_ref[:, rbase:rbase + 128] = acc1
        return beam

    beam = beam0
    for rbase in range(_T - 128, -1, -128):
        beam = chunk(rbase, beam)


def _decode_body(tok_ref, table_ref, out_ref):
    cidx = jax.lax.broadcasted_iota(jnp.int32, (1, _K), 1)
    oh = (tok_ref[...] == cidx).astype(jnp.float32)
    out_ref[...] = jnp.dot(oh, table_ref[...],
                           precision=jax.lax.Precision.HIGHEST,
                           preferred_element_type=jnp.float32)


def _run_encode(mixture, w_enc, boundaries):
    frames = mixture.reshape(_T, _HOP)
    w2 = w_enc.reshape(_HOP, 1)
    bnd = jnp.concatenate(
        [boundaries, jnp.full((1,), jnp.inf, jnp.float32)]).reshape(1, _K)
    codes2d = pl.pallas_call(
        _encode_body,
        out_shape=jax.ShapeDtypeStruct((_T, 128), jnp.int32),
        interpret=_INTERPRET,
    )(frames, w2, bnd)
    return codes2d[:, 0]


def _run_scan(codes, prior0, prior1, lik):
    return pl.pallas_call(
        _scan_body,
        grid_spec=pltpu.PrefetchScalarGridSpec(
            num_scalar_prefetch=1,
            grid=(_T,),
            in_specs=[
                pl.BlockSpec((_K, _K), lambda t, c: (0, 0)),
                pl.BlockSpec((_K, _K), lambda t, c: (0, 0)),
                pl.BlockSpec((1, _K, _K), lambda t, c: (c[t], 0, 0)),
            ],
            out_specs=[
                pl.BlockSpec((_T, 128), lambda t, c: (0, 0)),
                pl.BlockSpec((_T, 128), lambda t, c: (0, 0)),
                pl.BlockSpec((_T, 128), lambda t, c: (0, 0)),
            ],
            scratch_shapes=[
                pltpu.SMEM((_B,), jnp.float32),
                pltpu.SMEM((_B,), jnp.int32),
                pltpu.SMEM((_B,), jnp.int32),
                pltpu.VMEM((_B, _K), jnp.float32),
                pltpu.VMEM((_B, _K), jnp.float32),
            ],
        ),
        out_shape=[jax.ShapeDtypeStruct((_T, 128), jnp.int32)] * 3,
        interpret=_INTERPRET,
    )(codes, prior0, prior1, lik)


def _run_backtrack(bps, t0s, t1s):
    return pl.pallas_call(
        _backtrack_body,
        out_shape=[jax.ShapeDtypeStruct((_B, _T), jnp.int32)] * 2,
        interpret=_INTERPRET,
    )(bps, t0s, t1s)


def _run_decode(x0s, x1s, decode_table):
    toks = jnp.concatenate([x0s.reshape(-1), x1s.reshape(-1)]).reshape(-1, 1)
    n = toks.shape[0]
    dec = pl.pallas_call(
        _decode_body,
        grid=(n // 512,),
        in_specs=[
            pl.BlockSpec((512, 1), lambda g: (g, 0)),
            pl.BlockSpec((_K, _HOP), lambda g: (0, 0)),
        ],
        out_specs=pl.BlockSpec((512, _HOP), lambda g: (g, 0)),
        out_shape=jax.ShapeDtypeStruct((n, _HOP), jnp.float32),
        interpret=_INTERPRET,
    )(toks, decode_table)

    half = n // 2
    d0 = dec[:half].reshape(_B, _T * _HOP)
    d1 = dec[half:].reshape(_B, _T * _HOP)
    return (d0, d1)


def _sc_mesh():
    return plsc.VectorSubcoreMesh(core_axis_name="c", subcore_axis_name="s")


def _run_backtrack_sc(bps, t0s, t1s):
    bp = bps[:, :_B].reshape(-1)
    t0 = t0s[:, :_B].reshape(-1)
    t1 = t1s[:, :_B].reshape(-1)
    n = _B * _T
    npad = n + 16

    @functools.partial(
        pl.kernel, mesh=_sc_mesh(),
        out_type=[jax.ShapeDtypeStruct((16 * _T,), jnp.int32)] * 2,
        scratch_types=[pltpu.VMEM((npad,), jnp.int32)] * 3
        + [pltpu.VMEM((16 * _T,), jnp.int32)] * 2,
    )
    def bt(bp_hbm, t0_hbm, t1_hbm, x0_hbm, x1_hbm, bpv, t0v, t1v, x0v, x1v):
        wid = jax.lax.axis_index("s") * 2 + jax.lax.axis_index("c")

        @pl.when(wid == 0)
        def _():
            pltpu.sync_copy(bp_hbm, bpv.at[pl.ds(0, n)])
            pltpu.sync_copy(t0_hbm, t0v.at[pl.ds(0, n)])
            pltpu.sync_copy(t1_hbm, t1v.at[pl.ds(0, n)])
            lane = jax.lax.iota(jnp.int32, 16)
            msk = lane < _B
            beam0 = jnp.where(msk, lane, 0)

            def body(m, beam):
                r = (_T - 1) - m
                row0 = t0v[pl.ds(r * _B, 16)]
                row1 = t1v[pl.ds(r * _B, 16)]
                rowb = bpv[pl.ds(r * _B, 16)]
                x0v[pl.ds(r * 16, 16)] = row0.at[beam].get(
                    mode="promise_in_bounds")
                x1v[pl.ds(r * 16, 16)] = row1.at[beam].get(
                    mode="promise_in_bounds")
                nb = rowb.at[beam].get(mode="promise_in_bounds")
                return jnp.where(msk, nb, 0)

            jax.lax.fori_loop(0, _T, body, beam0, unroll=8)
            pltpu.sync_copy(x0v, x0_hbm)
            pltpu.sync_copy(x1v, x1_hbm)

    x0f, x1f = bt(bp, t0, t1)
    x0s = x0f.reshape(_T, 16)[:, :_B].T
    x1s = x1f.reshape(_T, 16)[:, :_B].T
    return x0s, x1s


def _run_decode_sc(x0s, x1s, decode_table):
    toks = jnp.concatenate([x0s.reshape(-1), x1s.reshape(-1)])
    n = toks.shape[0]
    nw = 32
    bpw = n // nw

    @functools.partial(
        pl.kernel, mesh=_sc_mesh(),
        out_type=jax.ShapeDtypeStruct((n, _HOP), jnp.float32),
        scratch_types=[
            pltpu.VMEM((bpw,), jnp.int32),
            pltpu.VMEM((bpw, _HOP), jnp.float32),
            pltpu.SemaphoreType.DMA,
        ],
    )
    def dec(tok_hbm, table_hbm, out_hbm, idx_v, rows_v, sem):
        wid = jax.lax.axis_index("s") * 2 + jax.lax.axis_index("c")
        base = wid * bpw
        pltpu.sync_copy(tok_hbm.at[pl.ds(base, bpw)], idx_v)
        pltpu.async_copy(table_hbm.at[idx_v], rows_v, sem).wait()
        pltpu.sync_copy(rows_v, out_hbm.at[pl.ds(base, bpw)])

    dec_out = dec(toks, decode_table)
    half = n // 2
    d0 = dec_out[:half].reshape(_B, _T * _HOP)
    d1 = dec_out[half:].reshape(_B, _T * _HOP)
    return (d0, d1)


def kernel(mixture, w_enc, boundaries, prior0, prior1, lik, decode_table):
    codes = _run_encode(mixture, w_enc, boundaries)
    bps, t0s, t1s = _run_scan(codes, prior0, prior1, lik)
    x0s, x1s = _run_backtrack_sc(bps, t0s, t1s)
    return _run_decode_sc(x0s, x1s, decode_table)
